# baseline (speedup 1.0000x reference)
import functools

import jax
import jax.numpy as jnp
from jax import lax
from jax.experimental import pallas as pl
from jax.experimental.pallas import tpu as pltpu
from jax.experimental.pallas import tpu_sc as plsc

NUM_ROWS = 1000000
FACTORS = 64
BATCH = 16384
LANES = 128

NC = 2
NS = 16
NW = NC * NS
B_PER_W = BATCH // NW

RANGE = 245 * LANES
PW = 512
NPIECE = RANGE // PW + 1
PSTART_MAX = 7811 * LANES
SELCAP = 1040
PLCAP = 80
PADROW = BATCH
IROWS = BATCH + 32


def _route_body(users_hbm, items_hbm, uT, iT, uv_out, iv_out,
                idxbuf, sel_lane, sel_pos, pl_lane, pl_pos,
                pieceA, pieceB, stag, ppos, semA, semB):
    wid = lax.axis_index("s") * NC + lax.axis_index("c")
    lo = wid * RANGE
    lanes16 = lax.iota(jnp.int32, 16)

    for idx_hbm, table, out in ((users_hbm, uT, uv_out),
                                (items_hbm, iT, iv_out)):
        pltpu.sync_copy(idx_hbm, idxbuf)

        def scan_body(t, ofs):
            vec = idxbuf[pl.ds(t * 16, 16)]
            m = (vec >= lo) & (vec < lo + RANGE)
            plsc.store_compressed(sel_lane.at[pl.ds(ofs, 16)], vec, mask=m)
            plsc.store_compressed(sel_pos.at[pl.ds(ofs, 16)],
                                  t * 16 + lanes16, mask=m)
            cnt = plsc.all_reduce_population_count(m)
            return ofs + cnt[0]

        nsel = lax.fori_loop(0, BATCH // 16, scan_body, 0)
        sel_lane[pl.ds(nsel, 16)] = jnp.full((16,), lo, jnp.int32)
        sel_pos[pl.ds(nsel, 16)] = PADROW + lanes16
        ngroups = (nsel + 15) >> 4

        def piece_start(pp):
            return jnp.minimum(lo + pp * PW, PSTART_MAX)

        def fire(pp, buf, sem):
            st = pl.multiple_of(piece_start(pp), LANES)
            return pltpu.async_copy(table.at[:, pl.ds(st, PW)], buf, sem)

        def process(pp, buf):
            start = piece_start(pp)

            def mbody(g, ofs2):
                lv = sel_lane[pl.ds(g * 16, 16)]
                pv = sel_pos[pl.ds(g * 16, 16)]
                m2 = (lv >= start) & (lv < start + PW)
                plsc.store_compressed(pl_lane.at[pl.ds(ofs2, 16)],
                                      lv - start, mask=m2)
                plsc.store_compressed(pl_pos.at[pl.ds(ofs2, 16)], pv, mask=m2)
                cnt = plsc.all_reduce_population_count(m2)
                return ofs2 + cnt[0]

            npc = lax.fori_loop(0, ngroups, mbody, 0)
            pl_lane[pl.ds(npc, 16)] = jnp.zeros((16,), jnp.int32)
            pl_pos[pl.ds(npc, 16)] = PADROW + lanes16

            def ebody(g, carry):
                ll = pl_lane[pl.ds(g * 16, 16)]
                qq = pl_pos[pl.ds(g * 16, 16)]
                ppos[0] = qq
                for e in range(16):
                    lu = jnp.full((16,), ll[e], jnp.int32)
                    for q in range(FACTORS // 16):
                        vreg = plsc.load_gather(
                            buf, [q * 16 + lanes16, lu])
                        stag[e, pl.ds(q * 16, 16)] = vreg
                pltpu.sync_copy(stag, out.at[ppos.at[0]])
                return carry

            lax.fori_loop(0, (npc + 15) >> 4, ebody, 0)

        def pair(t, carry):
            d0 = fire(2 * t, pieceA, semA)
            d1 = fire(2 * t + 1, pieceB, semB)
            d0.wait()
            process(2 * t, pieceA)
            d1.wait()
            process(2 * t + 1, pieceB)
            return carry

        lax.fori_loop(0, NPIECE // 2, pair, 0)


def _dot_body(uv, iv, out_hbm, ubuf, ibuf, pbuf, outv, semA):
    wid = lax.axis_index("s") * NC + lax.axis_index("c")
    base = wid * B_PER_W
    lanes16 = lax.iota(jnp.int32, 16)
    col0 = lanes16 * B_PER_W
    CH = 128

    def chunk(h, carry):
        r0 = h * CH
        pltpu.async_copy(uv.at[pl.ds(base + r0, CH), :], ubuf, semA).wait()
        pltpu.async_copy(iv.at[pl.ds(base + r0, CH), :], ibuf, semA).wait()

        def row(r, c2):
            s = jnp.zeros((16,), jnp.float32)
            for k in range(FACTORS // 16):
                u = ubuf[r, pl.ds(k * 16, 16)]
                v = ibuf[r, pl.ds(k * 16, 16)]
                s = s + u * v
            plsc.store_scatter(pbuf, [col0 + (r0 + r)], s)
            return c2

        lax.fori_loop(0, CH, row, 0, unroll=4)
        return carry

    lax.fori_loop(0, B_PER_W // CH, chunk, 0)

    def block(b, carry):
        acc = jnp.zeros((16,), jnp.float32)
        for l in range(16):
            acc = acc + pbuf[pl.ds(l * B_PER_W + b * 16, 16)]
        outv[pl.ds(b * 16, 16)] = acc
        return carry

    lax.fori_loop(0, B_PER_W // 16, block, 0)

    pltpu.sync_copy(outv, out_hbm.at[pl.ds(base, B_PER_W)])


@jax.jit
def _mf_dot(users, items, uT, iT):
    mesh = plsc.VectorSubcoreMesh(core_axis_name="c", subcore_axis_name="s")
    params = pltpu.CompilerParams(needs_layout_passes=False)
    uv, iv = pl.kernel(
        _route_body,
        mesh=mesh,
        compiler_params=params,
        out_type=[jax.ShapeDtypeStruct((IROWS, LANES), jnp.float32),
                  jax.ShapeDtypeStruct((IROWS, LANES), jnp.float32)],
        scratch_types=[
            pltpu.VMEM((BATCH,), jnp.int32),
            pltpu.VMEM((SELCAP,), jnp.int32),
            pltpu.VMEM((SELCAP,), jnp.int32),
            pltpu.VMEM((PLCAP,), jnp.int32),
            pltpu.VMEM((PLCAP,), jnp.int32),
            pltpu.VMEM((FACTORS, PW), jnp.float32),
            pltpu.VMEM((FACTORS, PW), jnp.float32),
            pltpu.VMEM((16, LANES), jnp.float32),
            pltpu.VMEM((1, 16), jnp.int32),
            pltpu.SemaphoreType.DMA,
            pltpu.SemaphoreType.DMA,
        ],
    )(users, items, uT, iT)

    return pl.kernel(
        _dot_body,
        mesh=mesh,
        compiler_params=params,
        out_type=jax.ShapeDtypeStruct((BATCH,), jnp.float32),
        scratch_types=[
            pltpu.VMEM((128, LANES), jnp.float32),
            pltpu.VMEM((128, LANES), jnp.float32),
            pltpu.VMEM((16 * B_PER_W,), jnp.float32),
            pltpu.VMEM((B_PER_W,), jnp.float32),
            pltpu.SemaphoreType.DMA,
        ],
    )(uv, iv)


def kernel(users, items, user_emb, item_emb):
    return _mf_dot(users.astype(jnp.int32), items.astype(jnp.int32),
                   user_emb.T, item_emb.T)


# speedup vs baseline: 1.3169x; 1.3169x over previous
import functools

import jax
import jax.numpy as jnp
from jax import lax
from jax.experimental import pallas as pl
from jax.experimental.pallas import tpu as pltpu
from jax.experimental.pallas import tpu_sc as plsc

NUM_ROWS = 1000000
FACTORS = 64
BATCH = 16384
LANES = 128

NC = 2
NS = 16
NW = NC * NS
B_PER_W = BATCH // NW

RANGE = 245 * LANES
PW = 512
SW = 4096
NSUPER = 8
PSTART_MAX = 7811 * LANES
SELCAP = 1040
SPLCAP = 288
PLCAP = 80
PADROW = BATCH
IROWS = BATCH + 128


def _route_body(users_hbm, items_hbm, uT, iT, uv_out, iv_out,
                idxbuf, sel_lane, sel_pos, spl_lane, spl_pos,
                pl_lane, pl_slot, pieceA, pieceB, stag, spos2d,
                semA, semB, semS):
    wid = lax.axis_index("s") * NC + lax.axis_index("c")
    lo = wid * RANGE
    lanes16 = lax.iota(jnp.int32, 16)

    for idx_hbm, table, out in ((users_hbm, uT, uv_out),
                                (items_hbm, iT, iv_out)):
        pltpu.sync_copy(idx_hbm, idxbuf)

        def scan_body(t, ofs):
            vec = idxbuf[pl.ds(t * 16, 16)]
            m = (vec >= lo) & (vec < lo + RANGE)
            plsc.store_compressed(sel_lane.at[pl.ds(ofs, 16)], vec, mask=m)
            plsc.store_compressed(sel_pos.at[pl.ds(ofs, 16)],
                                  t * 16 + lanes16, mask=m)
            cnt = plsc.all_reduce_population_count(m)
            return ofs + cnt[0]

        nsel = lax.fori_loop(0, BATCH // 16, scan_body, 0)
        sel_lane[pl.ds(nsel, 16)] = jnp.full((16,), lo, jnp.int32)
        sel_pos[pl.ds(nsel, 16)] = PADROW + lanes16
        ngroups = (nsel + 15) >> 4

        def fire(pp, buf, sem):
            st = pl.multiple_of(
                jnp.minimum(lo + pp * PW, PSTART_MAX), LANES)
            return pltpu.async_copy(table.at[:, pl.ds(st, PW)], buf, sem)

        def super_body(sp, carry):
            sstart = lo + sp * SW

            def smatch(g, ofs2):
                lv = sel_lane[pl.ds(g * 16, 16)]
                pv = sel_pos[pl.ds(g * 16, 16)]
                m2 = (lv >= sstart) & (lv < sstart + SW)
                plsc.store_compressed(spl_lane.at[pl.ds(ofs2, 16)],
                                      lv - sstart, mask=m2)
                plsc.store_compressed(spl_pos.at[pl.ds(ofs2, 16)],
                                      pv, mask=m2)
                cnt = plsc.all_reduce_population_count(m2)
                return ofs2 + cnt[0]

            nsp = lax.fori_loop(0, ngroups, smatch, 0)
            spl_lane[pl.ds(nsp, 16)] = jnp.zeros((16,), jnp.int32)
            spl_pos[pl.ds(nsp, 16)] = PADROW + lanes16
            sgroups = (nsp + 15) >> 4

            for c in range(2):
                for q in range(8):
                    spos2d[c, pl.ds(q * 16, 16)] = PADROW + q * 16 + lanes16

            def posfill(g, carry2):
                spos2d[g >> 3, pl.ds((g & 7) * 16, 16)] = \
                    spl_pos[pl.ds(g * 16, 16)]
                return carry2

            lax.fori_loop(0, sgroups, posfill, 0)

            def process(pp, buf):
                rel = jnp.minimum(lo + pp * PW, PSTART_MAX) - sstart

                def mbody(g, ofs2):
                    lv = spl_lane[pl.ds(g * 16, 16)]
                    m2 = (lv >= rel) & (lv < rel + PW)
                    plsc.store_compressed(pl_lane.at[pl.ds(ofs2, 16)],
                                          lv - rel, mask=m2)
                    plsc.store_compressed(pl_slot.at[pl.ds(ofs2, 16)],
                                          g * 16 + lanes16, mask=m2)
                    cnt = plsc.all_reduce_population_count(m2)
                    return ofs2 + cnt[0]

                npc = lax.fori_loop(0, sgroups, mbody, 0)
                pl_lane[pl.ds(npc, 16)] = jnp.zeros((16,), jnp.int32)
                pl_slot[pl.ds(npc, 16)] = (SPLCAP - 16) + lanes16

                def ebody(g, carry2):
                    ll = pl_lane[pl.ds(g * 16, 16)]
                    ss = pl_slot[pl.ds(g * 16, 16)]
                    for e in range(16):
                        lu = jnp.full((16,), ll[e], jnp.int32)
                        slot = ss[e]
                        for q in range(FACTORS // 16):
                            vreg = plsc.load_gather(
                                buf, [q * 16 + lanes16, lu])
                            stag[slot, pl.ds(q * 16, 16)] = vreg
                    return carry2

                lax.fori_loop(0, (npc + 15) >> 4, ebody, 0)

            for k in range(4):
                pp0 = sp * 8 + 2 * k
                d0 = fire(pp0, pieceA, semA)
                d1 = fire(pp0 + 1, pieceB, semB)
                d0.wait()
                process(pp0, pieceA)
                d1.wait()
                process(pp0 + 1, pieceB)

            s0 = pltpu.async_copy(stag.at[pl.ds(0, 128), :],
                                  out.at[spos2d.at[0]], semS)
            s1 = pltpu.async_copy(stag.at[pl.ds(128, 128), :],
                                  out.at[spos2d.at[1]], semS)
            s0.wait()
            s1.wait()
            return carry

        lax.fori_loop(0, NSUPER, super_body, 0)


def _dot_body(uv, iv, out_hbm, ubuf, ibuf, pbuf, outv, semA):
    wid = lax.axis_index("s") * NC + lax.axis_index("c")
    base = wid * B_PER_W
    lanes16 = lax.iota(jnp.int32, 16)
    col0 = lanes16 * B_PER_W
    CH = 128

    def chunk(h, carry):
        r0 = h * CH
        pltpu.async_copy(uv.at[pl.ds(base + r0, CH), :], ubuf, semA).wait()
        pltpu.async_copy(iv.at[pl.ds(base + r0, CH), :], ibuf, semA).wait()

        def row(r, c2):
            s = jnp.zeros((16,), jnp.float32)
            for k in range(FACTORS // 16):
                u = ubuf[r, pl.ds(k * 16, 16)]
                v = ibuf[r, pl.ds(k * 16, 16)]
                s = s + u * v
            plsc.store_scatter(pbuf, [col0 + (r0 + r)], s)
            return c2

        lax.fori_loop(0, CH, row, 0, unroll=4)
        return carry

    lax.fori_loop(0, B_PER_W // CH, chunk, 0)

    def block(b, carry):
        acc = jnp.zeros((16,), jnp.float32)
        for l in range(16):
            acc = acc + pbuf[pl.ds(l * B_PER_W + b * 16, 16)]
        outv[pl.ds(b * 16, 16)] = acc
        return carry

    lax.fori_loop(0, B_PER_W // 16, block, 0)

    pltpu.sync_copy(outv, out_hbm.at[pl.ds(base, B_PER_W)])


@jax.jit
def _mf_dot(users, items, uT, iT):
    mesh = plsc.VectorSubcoreMesh(core_axis_name="c", subcore_axis_name="s")
    params = pltpu.CompilerParams(needs_layout_passes=False)
    uv, iv = pl.kernel(
        _route_body,
        mesh=mesh,
        compiler_params=params,
        out_type=[jax.ShapeDtypeStruct((IROWS, LANES), jnp.float32),
                  jax.ShapeDtypeStruct((IROWS, LANES), jnp.float32)],
        scratch_types=[
            pltpu.VMEM((BATCH,), jnp.int32),
            pltpu.VMEM((SELCAP,), jnp.int32),
            pltpu.VMEM((SELCAP,), jnp.int32),
            pltpu.VMEM((SPLCAP,), jnp.int32),
            pltpu.VMEM((SPLCAP,), jnp.int32),
            pltpu.VMEM((PLCAP,), jnp.int32),
            pltpu.VMEM((PLCAP,), jnp.int32),
            pltpu.VMEM((FACTORS, PW), jnp.float32),
            pltpu.VMEM((FACTORS, PW), jnp.float32),
            pltpu.VMEM((SPLCAP, LANES), jnp.float32),
            pltpu.VMEM((2, 128), jnp.int32),
            pltpu.SemaphoreType.DMA,
            pltpu.SemaphoreType.DMA,
            pltpu.SemaphoreType.DMA,
        ],
    )(users, items, uT, iT)

    return pl.kernel(
        _dot_body,
        mesh=mesh,
        compiler_params=params,
        out_type=jax.ShapeDtypeStruct((BATCH,), jnp.float32),
        scratch_types=[
            pltpu.VMEM((128, LANES), jnp.float32),
            pltpu.VMEM((128, LANES), jnp.float32),
            pltpu.VMEM((16 * B_PER_W,), jnp.float32),
            pltpu.VMEM((B_PER_W,), jnp.float32),
            pltpu.SemaphoreType.DMA,
        ],
    )(uv, iv)


def kernel(users, items, user_emb, item_emb):
    return _mf_dot(users.astype(jnp.int32), items.astype(jnp.int32),
                   user_emb.T, item_emb.T)


# speedup vs baseline: 1.4370x; 1.0912x over previous
import functools

import jax
import jax.numpy as jnp
from jax import lax
from jax.experimental import pallas as pl
from jax.experimental.pallas import tpu as pltpu
from jax.experimental.pallas import tpu_sc as plsc

NUM_ROWS = 1000000
FACTORS = 64
BATCH = 16384
LANES = 128

NC = 2
NS = 16
NW = NC * NS
B_PER_W = BATCH // NW

RANGE = 245 * LANES
PW = 512
SW = 4096
NSUPER = 8
PSTART_MAX = 7811 * LANES
SELCAP = 1040
SPLCAP = 288
PLCAP = 80
PADROW = BATCH
IROWS = BATCH + 128


def _route_body(users_hbm, items_hbm, uT, iT, uv_out, iv_out,
                idxbuf, sel_lane, sel_pos, spl_lane, spl_pos,
                pl_lane, pl_slot, pieceA, pieceB, stag, spos2d,
                semA, semB, semS):
    wid = lax.axis_index("s") * NC + lax.axis_index("c")
    lo = wid * RANGE
    lanes16 = lax.iota(jnp.int32, 16)

    for idx_hbm, table, out in ((users_hbm, uT, uv_out),
                                (items_hbm, iT, iv_out)):
        pltpu.sync_copy(idx_hbm, idxbuf)

        def scan_body(t, ofs):
            vec = idxbuf[pl.ds(t * 16, 16)]
            m = (vec >= lo) & (vec < lo + RANGE)
            plsc.store_compressed(sel_lane.at[pl.ds(ofs, 16)], vec, mask=m)
            plsc.store_compressed(sel_pos.at[pl.ds(ofs, 16)],
                                  t * 16 + lanes16, mask=m)
            cnt = plsc.all_reduce_population_count(m)
            return ofs + cnt[0]

        nsel = lax.fori_loop(0, BATCH // 16, scan_body, 0)
        sel_lane[pl.ds(nsel, 16)] = jnp.full((16,), lo, jnp.int32)
        sel_pos[pl.ds(nsel, 16)] = PADROW + lanes16
        ngroups = (nsel + 15) >> 4

        def fire(pp, buf, sem):
            st = pl.multiple_of(
                jnp.minimum(lo + pp * PW, PSTART_MAX), LANES)
            return pltpu.async_copy(table.at[:, pl.ds(st, PW)], buf, sem)

        def super_body(sp, carry):
            sstart = lo + sp * SW

            def smatch(g, ofs2):
                lv = sel_lane[pl.ds(g * 16, 16)]
                pv = sel_pos[pl.ds(g * 16, 16)]
                m2 = (lv >= sstart) & (lv < sstart + SW)
                plsc.store_compressed(spl_lane.at[pl.ds(ofs2, 16)],
                                      lv - sstart, mask=m2)
                plsc.store_compressed(spl_pos.at[pl.ds(ofs2, 16)],
                                      pv, mask=m2)
                cnt = plsc.all_reduce_population_count(m2)
                return ofs2 + cnt[0]

            nsp = lax.fori_loop(0, ngroups, smatch, 0)
            spl_lane[pl.ds(nsp, 16)] = jnp.zeros((16,), jnp.int32)
            spl_pos[pl.ds(nsp, 16)] = PADROW + lanes16
            sgroups = (nsp + 15) >> 4

            for c in range(2):
                for q in range(8):
                    spos2d[c, pl.ds(q * 16, 16)] = PADROW + q * 16 + lanes16

            def posfill(g, carry2):
                spos2d[g >> 3, pl.ds((g & 7) * 16, 16)] = \
                    spl_pos[pl.ds(g * 16, 16)]
                return carry2

            lax.fori_loop(0, sgroups, posfill, 0)

            def process(pp, buf):
                rel = jnp.minimum(lo + pp * PW, PSTART_MAX) - sstart

                def mbody(g, ofs2):
                    lv = spl_lane[pl.ds(g * 16, 16)]
                    m2 = (lv >= rel) & (lv < rel + PW)
                    plsc.store_compressed(pl_lane.at[pl.ds(ofs2, 16)],
                                          lv - rel, mask=m2)
                    plsc.store_compressed(pl_slot.at[pl.ds(ofs2, 16)],
                                          g * 16 + lanes16, mask=m2)
                    cnt = plsc.all_reduce_population_count(m2)
                    return ofs2 + cnt[0]

                npc = lax.fori_loop(0, sgroups, mbody, 0)
                pl_lane[pl.ds(npc, 16)] = jnp.zeros((16,), jnp.int32)
                pl_slot[pl.ds(npc, 16)] = (SPLCAP - 16) + lanes16

                def ebody(g, carry2):
                    ll = pl_lane[pl.ds(g * 16, 16)]
                    ss = pl_slot[pl.ds(g * 16, 16)]
                    for e in range(16):
                        lu = jnp.full((16,), ll[e], jnp.int32)
                        slot = ss[e]
                        for q in range(FACTORS // 16):
                            vreg = plsc.load_gather(
                                buf, [q * 16 + lanes16, lu])
                            stag[slot, pl.ds(q * 16, 16)] = vreg
                    return carry2

                lax.fori_loop(0, (npc + 15) >> 4, ebody, 0)

            bufs = (pieceA, pieceB)
            sems = (semA, semB)
            descs = [None] * 8
            descs[0] = fire(sp * 8, pieceA, semA)
            for pc in range(8):
                if pc < 7:
                    descs[pc + 1] = fire(sp * 8 + pc + 1,
                                         bufs[(pc + 1) % 2], sems[(pc + 1) % 2])
                descs[pc].wait()
                process(sp * 8 + pc, bufs[pc % 2])

            s0 = pltpu.async_copy(stag.at[pl.ds(0, 128), :],
                                  out.at[spos2d.at[0]], semS)
            s1 = pltpu.async_copy(stag.at[pl.ds(128, 128), :],
                                  out.at[spos2d.at[1]], semS)
            s0.wait()
            s1.wait()
            return carry

        lax.fori_loop(0, NSUPER, super_body, 0)


def _dot_body(uv, iv, out_hbm, ubuf, ibuf, pbuf, outv, semA):
    wid = lax.axis_index("s") * NC + lax.axis_index("c")
    base = wid * B_PER_W
    lanes16 = lax.iota(jnp.int32, 16)
    col0 = lanes16 * B_PER_W
    CH = 128

    def chunk(h, carry):
        r0 = h * CH
        pltpu.async_copy(uv.at[pl.ds(base + r0, CH), :], ubuf, semA).wait()
        pltpu.async_copy(iv.at[pl.ds(base + r0, CH), :], ibuf, semA).wait()

        def row(r, c2):
            s = jnp.zeros((16,), jnp.float32)
            for k in range(FACTORS // 16):
                u = ubuf[r, pl.ds(k * 16, 16)]
                v = ibuf[r, pl.ds(k * 16, 16)]
                s = s + u * v
            plsc.store_scatter(pbuf, [col0 + (r0 + r)], s)
            return c2

        lax.fori_loop(0, CH, row, 0, unroll=4)
        return carry

    lax.fori_loop(0, B_PER_W // CH, chunk, 0)

    def block(b, carry):
        acc = jnp.zeros((16,), jnp.float32)
        for l in range(16):
            acc = acc + pbuf[pl.ds(l * B_PER_W + b * 16, 16)]
        outv[pl.ds(b * 16, 16)] = acc
        return carry

    lax.fori_loop(0, B_PER_W // 16, block, 0)

    pltpu.sync_copy(outv, out_hbm.at[pl.ds(base, B_PER_W)])


@jax.jit
def _mf_dot(users, items, uT, iT):
    mesh = plsc.VectorSubcoreMesh(core_axis_name="c", subcore_axis_name="s")
    params = pltpu.CompilerParams(needs_layout_passes=False)
    uv, iv = pl.kernel(
        _route_body,
        mesh=mesh,
        compiler_params=params,
        out_type=[jax.ShapeDtypeStruct((IROWS, LANES), jnp.float32),
                  jax.ShapeDtypeStruct((IROWS, LANES), jnp.float32)],
        scratch_types=[
            pltpu.VMEM((BATCH,), jnp.int32),
            pltpu.VMEM((SELCAP,), jnp.int32),
            pltpu.VMEM((SELCAP,), jnp.int32),
            pltpu.VMEM((SPLCAP,), jnp.int32),
            pltpu.VMEM((SPLCAP,), jnp.int32),
            pltpu.VMEM((PLCAP,), jnp.int32),
            pltpu.VMEM((PLCAP,), jnp.int32),
            pltpu.VMEM((FACTORS, PW), jnp.float32),
            pltpu.VMEM((FACTORS, PW), jnp.float32),
            pltpu.VMEM((SPLCAP, LANES), jnp.float32),
            pltpu.VMEM((2, 128), jnp.int32),
            pltpu.SemaphoreType.DMA,
            pltpu.SemaphoreType.DMA,
            pltpu.SemaphoreType.DMA,
        ],
    )(users, items, uT, iT)

    return pl.kernel(
        _dot_body,
        mesh=mesh,
        compiler_params=params,
        out_type=jax.ShapeDtypeStruct((BATCH,), jnp.float32),
        scratch_types=[
            pltpu.VMEM((128, LANES), jnp.float32),
            pltpu.VMEM((128, LANES), jnp.float32),
            pltpu.VMEM((16 * B_PER_W,), jnp.float32),
            pltpu.VMEM((B_PER_W,), jnp.float32),
            pltpu.SemaphoreType.DMA,
        ],
    )(uv, iv)


def kernel(users, items, user_emb, item_emb):
    return _mf_dot(users.astype(jnp.int32), items.astype(jnp.int32),
                   user_emb.T, item_emb.T)


# speedup vs baseline: 1.4442x; 1.0050x over previous
import functools

import jax
import jax.numpy as jnp
from jax import lax
from jax.experimental import pallas as pl
from jax.experimental.pallas import tpu as pltpu
from jax.experimental.pallas import tpu_sc as plsc

NUM_ROWS = 1000000
FACTORS = 64
BATCH = 16384
LANES = 128

NC = 2
NS = 16
NW = NC * NS
B_PER_W = BATCH // NW

RANGE = 245 * LANES
PW = 512
SW = 4096
NSUPER = 8
PSTART_MAX = 7811 * LANES
SELCAP = 1040
SPLCAP = 288
PLCAP = 80
PADROW = BATCH
IROWS = BATCH + 128


def _route_body(users_hbm, items_hbm, uT, iT, uv_out, iv_out,
                idxbuf, sel_lane, sel_pos, spl_lane, spl_pos,
                pl_lane, pl_slot, pieceA, pieceB, stag, spos2d,
                semA, semB, semS):
    wid = lax.axis_index("s") * NC + lax.axis_index("c")
    lo = wid * RANGE
    lanes16 = lax.iota(jnp.int32, 16)

    for idx_hbm, table, out in ((users_hbm, uT, uv_out),
                                (items_hbm, iT, iv_out)):
        pltpu.sync_copy(idx_hbm, idxbuf)

        def scan_body(t, ofs):
            vec = idxbuf[pl.ds(t * 16, 16)]
            m = (vec >= lo) & (vec < lo + RANGE)
            plsc.store_compressed(sel_lane.at[pl.ds(ofs, 16)], vec, mask=m)
            plsc.store_compressed(sel_pos.at[pl.ds(ofs, 16)],
                                  t * 16 + lanes16, mask=m)
            cnt = plsc.all_reduce_population_count(m)
            return ofs + cnt[0]

        nsel = lax.fori_loop(0, BATCH // 16, scan_body, 0, unroll=4)
        sel_lane[pl.ds(nsel, 16)] = jnp.full((16,), lo, jnp.int32)
        sel_pos[pl.ds(nsel, 16)] = PADROW + lanes16
        ngroups = (nsel + 15) >> 4

        def fire(pp, buf, sem):
            st = pl.multiple_of(
                jnp.minimum(lo + pp * PW, PSTART_MAX), LANES)
            return pltpu.async_copy(table.at[:, pl.ds(st, PW)], buf, sem)

        def super_body(sp, carry):
            sstart = lo + sp * SW

            def smatch(g, ofs2):
                lv = sel_lane[pl.ds(g * 16, 16)]
                pv = sel_pos[pl.ds(g * 16, 16)]
                m2 = (lv >= sstart) & (lv < sstart + SW)
                plsc.store_compressed(spl_lane.at[pl.ds(ofs2, 16)],
                                      lv - sstart, mask=m2)
                plsc.store_compressed(spl_pos.at[pl.ds(ofs2, 16)],
                                      pv, mask=m2)
                cnt = plsc.all_reduce_population_count(m2)
                return ofs2 + cnt[0]

            nsp = lax.fori_loop(0, ngroups, smatch, 0)
            spl_lane[pl.ds(nsp, 16)] = jnp.zeros((16,), jnp.int32)
            spl_pos[pl.ds(nsp, 16)] = PADROW + lanes16
            sgroups = (nsp + 15) >> 4

            for c in range(2):
                for q in range(8):
                    spos2d[c, pl.ds(q * 16, 16)] = PADROW + q * 16 + lanes16

            def posfill(g, carry2):
                spos2d[g >> 3, pl.ds((g & 7) * 16, 16)] = \
                    spl_pos[pl.ds(g * 16, 16)]
                return carry2

            lax.fori_loop(0, sgroups, posfill, 0)

            def process(pp, buf):
                rel = jnp.minimum(lo + pp * PW, PSTART_MAX) - sstart

                def mbody(g, ofs2):
                    lv = spl_lane[pl.ds(g * 16, 16)]
                    m2 = (lv >= rel) & (lv < rel + PW)
                    plsc.store_compressed(pl_lane.at[pl.ds(ofs2, 16)],
                                          lv - rel, mask=m2)
                    plsc.store_compressed(pl_slot.at[pl.ds(ofs2, 16)],
                                          g * 16 + lanes16, mask=m2)
                    cnt = plsc.all_reduce_population_count(m2)
                    return ofs2 + cnt[0]

                npc = lax.fori_loop(0, sgroups, mbody, 0)
                pl_lane[pl.ds(npc, 16)] = jnp.zeros((16,), jnp.int32)
                pl_slot[pl.ds(npc, 16)] = (SPLCAP - 16) + lanes16

                def ebody(g, carry2):
                    ll = pl_lane[pl.ds(g * 16, 16)]
                    ss = pl_slot[pl.ds(g * 16, 16)]
                    for e in range(16):
                        lu = jnp.full((16,), ll[e], jnp.int32)
                        slot = ss[e]
                        for q in range(FACTORS // 16):
                            vreg = plsc.load_gather(
                                buf, [q * 16 + lanes16, lu])
                            stag[slot, pl.ds(q * 16, 16)] = vreg
                    return carry2

                lax.fori_loop(0, (npc + 15) >> 4, ebody, 0)

            bufs = (pieceA, pieceB)
            sems = (semA, semB)
            descs = [None] * 8
            descs[0] = fire(sp * 8, pieceA, semA)
            for pc in range(8):
                if pc < 7:
                    descs[pc + 1] = fire(sp * 8 + pc + 1,
                                         bufs[(pc + 1) % 2], sems[(pc + 1) % 2])
                descs[pc].wait()
                process(sp * 8 + pc, bufs[pc % 2])

            s0 = pltpu.async_copy(stag.at[pl.ds(0, 128), :],
                                  out.at[spos2d.at[0]], semS)
            s1 = pltpu.async_copy(stag.at[pl.ds(128, 128), :],
                                  out.at[spos2d.at[1]], semS)
            s0.wait()
            s1.wait()
            return carry

        lax.fori_loop(0, NSUPER, super_body, 0)


def _dot_body(uv, iv, out_hbm, ubuf, ibuf, pbuf, outv, semA):
    wid = lax.axis_index("s") * NC + lax.axis_index("c")
    base = wid * B_PER_W
    lanes16 = lax.iota(jnp.int32, 16)
    col0 = lanes16 * B_PER_W
    CH = 128

    def chunk(h, carry):
        r0 = h * CH
        pltpu.async_copy(uv.at[pl.ds(base + r0, CH), :], ubuf, semA).wait()
        pltpu.async_copy(iv.at[pl.ds(base + r0, CH), :], ibuf, semA).wait()

        def row(r, c2):
            s = jnp.zeros((16,), jnp.float32)
            for k in range(FACTORS // 16):
                u = ubuf[r, pl.ds(k * 16, 16)]
                v = ibuf[r, pl.ds(k * 16, 16)]
                s = s + u * v
            plsc.store_scatter(pbuf, [col0 + (r0 + r)], s)
            return c2

        lax.fori_loop(0, CH, row, 0, unroll=4)
        return carry

    lax.fori_loop(0, B_PER_W // CH, chunk, 0)

    def block(b, carry):
        acc = jnp.zeros((16,), jnp.float32)
        for l in range(16):
            acc = acc + pbuf[pl.ds(l * B_PER_W + b * 16, 16)]
        outv[pl.ds(b * 16, 16)] = acc
        return carry

    lax.fori_loop(0, B_PER_W // 16, block, 0)

    pltpu.sync_copy(outv, out_hbm.at[pl.ds(base, B_PER_W)])


@jax.jit
def _mf_dot(users, items, uT, iT):
    mesh = plsc.VectorSubcoreMesh(core_axis_name="c", subcore_axis_name="s")
    params = pltpu.CompilerParams(needs_layout_passes=False)
    uv, iv = pl.kernel(
        _route_body,
        mesh=mesh,
        compiler_params=params,
        out_type=[jax.ShapeDtypeStruct((IROWS, LANES), jnp.float32),
                  jax.ShapeDtypeStruct((IROWS, LANES), jnp.float32)],
        scratch_types=[
            pltpu.VMEM((BATCH,), jnp.int32),
            pltpu.VMEM((SELCAP,), jnp.int32),
            pltpu.VMEM((SELCAP,), jnp.int32),
            pltpu.VMEM((SPLCAP,), jnp.int32),
            pltpu.VMEM((SPLCAP,), jnp.int32),
            pltpu.VMEM((PLCAP,), jnp.int32),
            pltpu.VMEM((PLCAP,), jnp.int32),
            pltpu.VMEM((FACTORS, PW), jnp.float32),
            pltpu.VMEM((FACTORS, PW), jnp.float32),
            pltpu.VMEM((SPLCAP, LANES), jnp.float32),
            pltpu.VMEM((2, 128), jnp.int32),
            pltpu.SemaphoreType.DMA,
            pltpu.SemaphoreType.DMA,
            pltpu.SemaphoreType.DMA,
        ],
    )(users, items, uT, iT)

    return pl.kernel(
        _dot_body,
        mesh=mesh,
        compiler_params=params,
        out_type=jax.ShapeDtypeStruct((BATCH,), jnp.float32),
        scratch_types=[
            pltpu.VMEM((128, LANES), jnp.float32),
            pltpu.VMEM((128, LANES), jnp.float32),
            pltpu.VMEM((16 * B_PER_W,), jnp.float32),
            pltpu.VMEM((B_PER_W,), jnp.float32),
            pltpu.SemaphoreType.DMA,
        ],
    )(uv, iv)


def kernel(users, items, user_emb, item_emb):
    return _mf_dot(users.astype(jnp.int32), items.astype(jnp.int32),
                   user_emb.T, item_emb.T)


# speedup vs baseline: 1.4495x; 1.0037x over previous
import functools

import jax
import jax.numpy as jnp
from jax import lax
from jax.experimental import pallas as pl
from jax.experimental.pallas import tpu as pltpu
from jax.experimental.pallas import tpu_sc as plsc

NUM_ROWS = 1000000
FACTORS = 64
BATCH = 16384
LANES = 128

NC = 2
NS = 16
NW = NC * NS
B_PER_W = BATCH // NW

RANGE = 245 * LANES
PW = 512
SW = 4096
NSUPER = 8
PSTART_MAX = 7811 * LANES
SELCAP = 1040
SPLCAP = 288
PLCAP = 80
PADROW = BATCH
IROWS = BATCH + 128


def _route_body(users_hbm, items_hbm, uT, iT, uv_out, iv_out,
                idxbuf, sel_lane, sel_pos, spl_lane, spl_pos,
                pl_lane, pl_slot, pieceA, pieceB, stag, spos2d,
                semA, semB, semS):
    wid = lax.axis_index("s") * NC + lax.axis_index("c")
    lo = wid * RANGE
    lanes16 = lax.iota(jnp.int32, 16)

    for idx_hbm, table, out in ((users_hbm, uT, uv_out),
                                (items_hbm, iT, iv_out)):
        pltpu.sync_copy(idx_hbm, idxbuf)

        def scan_body(t, ofs):
            vec = idxbuf[pl.ds(t * 16, 16)]
            m = (vec >= lo) & (vec < lo + RANGE)
            plsc.store_compressed(sel_lane.at[pl.ds(ofs, 16)], vec, mask=m)
            plsc.store_compressed(sel_pos.at[pl.ds(ofs, 16)],
                                  t * 16 + lanes16, mask=m)
            cnt = plsc.all_reduce_population_count(m)
            return ofs + cnt[0]

        nsel = lax.fori_loop(0, BATCH // 16, scan_body, 0, unroll=4)
        sel_lane[pl.ds(nsel, 16)] = jnp.full((16,), lo, jnp.int32)
        sel_pos[pl.ds(nsel, 16)] = PADROW + lanes16
        ngroups = (nsel + 15) >> 4

        def fire(pp, buf, sem):
            st = pl.multiple_of(
                jnp.minimum(lo + pp * PW, PSTART_MAX), LANES)
            return pltpu.async_copy(table.at[:, pl.ds(st, PW)], buf, sem)

        def super_body(sp, carry):
            sstart = lo + sp * SW

            def smatch(g, ofs2):
                lv = sel_lane[pl.ds(g * 16, 16)]
                pv = sel_pos[pl.ds(g * 16, 16)]
                m2 = (lv >= sstart) & (lv < sstart + SW)
                plsc.store_compressed(spl_lane.at[pl.ds(ofs2, 16)],
                                      lv - sstart, mask=m2)
                plsc.store_compressed(spl_pos.at[pl.ds(ofs2, 16)],
                                      pv, mask=m2)
                cnt = plsc.all_reduce_population_count(m2)
                return ofs2 + cnt[0]

            nsp = lax.fori_loop(0, ngroups, smatch, 0)
            spl_lane[pl.ds(nsp, 16)] = jnp.zeros((16,), jnp.int32)
            spl_pos[pl.ds(nsp, 16)] = PADROW + lanes16
            sgroups = (nsp + 15) >> 4

            for c in range(2):
                for q in range(8):
                    spos2d[c, pl.ds(q * 16, 16)] = PADROW + q * 16 + lanes16

            def posfill(g, carry2):
                spos2d[g >> 3, pl.ds((g & 7) * 16, 16)] = \
                    spl_pos[pl.ds(g * 16, 16)]
                return carry2

            lax.fori_loop(0, sgroups, posfill, 0)

            def process(pp, buf):
                rel = jnp.minimum(lo + pp * PW, PSTART_MAX) - sstart

                def mbody(g, ofs2):
                    lv = spl_lane[pl.ds(g * 16, 16)]
                    m2 = (lv >= rel) & (lv < rel + PW)
                    plsc.store_compressed(pl_lane.at[pl.ds(ofs2, 16)],
                                          lv - rel, mask=m2)
                    plsc.store_compressed(pl_slot.at[pl.ds(ofs2, 16)],
                                          g * 16 + lanes16, mask=m2)
                    cnt = plsc.all_reduce_population_count(m2)
                    return ofs2 + cnt[0]

                npc = lax.fori_loop(0, sgroups, mbody, 0)
                pl_lane[pl.ds(npc, 16)] = jnp.zeros((16,), jnp.int32)
                pl_slot[pl.ds(npc, 16)] = (SPLCAP - 16) + lanes16

                def ebody(g, carry2):
                    ll = pl_lane[pl.ds(g * 16, 16)]
                    ss = pl_slot[pl.ds(g * 16, 16)]
                    for f in range(FACTORS):
                        fvec = jnp.full((16,), f, jnp.int32)
                        vals = plsc.load_gather(buf, [fvec, ll])
                        plsc.store_scatter(stag, [ss, fvec], vals)
                    return carry2

                lax.fori_loop(0, (npc + 15) >> 4, ebody, 0)

            bufs = (pieceA, pieceB)
            sems = (semA, semB)
            descs = [None] * 8
            descs[0] = fire(sp * 8, pieceA, semA)
            for pc in range(8):
                if pc < 7:
                    descs[pc + 1] = fire(sp * 8 + pc + 1,
                                         bufs[(pc + 1) % 2], sems[(pc + 1) % 2])
                descs[pc].wait()
                process(sp * 8 + pc, bufs[pc % 2])

            s0 = pltpu.async_copy(stag.at[pl.ds(0, 128), :],
                                  out.at[spos2d.at[0]], semS)
            s1 = pltpu.async_copy(stag.at[pl.ds(128, 128), :],
                                  out.at[spos2d.at[1]], semS)
            s0.wait()
            s1.wait()
            return carry

        lax.fori_loop(0, NSUPER, super_body, 0)


def _dot_body(uv, iv, out_hbm, ubuf, ibuf, pbuf, outv, semA):
    wid = lax.axis_index("s") * NC + lax.axis_index("c")
    base = wid * B_PER_W
    lanes16 = lax.iota(jnp.int32, 16)
    col0 = lanes16 * B_PER_W
    CH = 128

    def chunk(h, carry):
        r0 = h * CH
        pltpu.async_copy(uv.at[pl.ds(base + r0, CH), :], ubuf, semA).wait()
        pltpu.async_copy(iv.at[pl.ds(base + r0, CH), :], ibuf, semA).wait()

        def row(r, c2):
            s = jnp.zeros((16,), jnp.float32)
            for k in range(FACTORS // 16):
                u = ubuf[r, pl.ds(k * 16, 16)]
                v = ibuf[r, pl.ds(k * 16, 16)]
                s = s + u * v
            plsc.store_scatter(pbuf, [col0 + (r0 + r)], s)
            return c2

        lax.fori_loop(0, CH, row, 0, unroll=4)
        return carry

    lax.fori_loop(0, B_PER_W // CH, chunk, 0)

    def block(b, carry):
        acc = jnp.zeros((16,), jnp.float32)
        for l in range(16):
            acc = acc + pbuf[pl.ds(l * B_PER_W + b * 16, 16)]
        outv[pl.ds(b * 16, 16)] = acc
        return carry

    lax.fori_loop(0, B_PER_W // 16, block, 0)

    pltpu.sync_copy(outv, out_hbm.at[pl.ds(base, B_PER_W)])


@jax.jit
def _mf_dot(users, items, uT, iT):
    mesh = plsc.VectorSubcoreMesh(core_axis_name="c", subcore_axis_name="s")
    params = pltpu.CompilerParams(needs_layout_passes=False)
    uv, iv = pl.kernel(
        _route_body,
        mesh=mesh,
        compiler_params=params,
        out_type=[jax.ShapeDtypeStruct((IROWS, LANES), jnp.float32),
                  jax.ShapeDtypeStruct((IROWS, LANES), jnp.float32)],
        scratch_types=[
            pltpu.VMEM((BATCH,), jnp.int32),
            pltpu.VMEM((SELCAP,), jnp.int32),
            pltpu.VMEM((SELCAP,), jnp.int32),
            pltpu.VMEM((SPLCAP,), jnp.int32),
            pltpu.VMEM((SPLCAP,), jnp.int32),
            pltpu.VMEM((PLCAP,), jnp.int32),
            pltpu.VMEM((PLCAP,), jnp.int32),
            pltpu.VMEM((FACTORS, PW), jnp.float32),
            pltpu.VMEM((FACTORS, PW), jnp.float32),
            pltpu.VMEM((SPLCAP, LANES), jnp.float32),
            pltpu.VMEM((2, 128), jnp.int32),
            pltpu.SemaphoreType.DMA,
            pltpu.SemaphoreType.DMA,
            pltpu.SemaphoreType.DMA,
        ],
    )(users, items, uT, iT)

    return pl.kernel(
        _dot_body,
        mesh=mesh,
        compiler_params=params,
        out_type=jax.ShapeDtypeStruct((BATCH,), jnp.float32),
        scratch_types=[
            pltpu.VMEM((128, LANES), jnp.float32),
            pltpu.VMEM((128, LANES), jnp.float32),
            pltpu.VMEM((16 * B_PER_W,), jnp.float32),
            pltpu.VMEM((B_PER_W,), jnp.float32),
            pltpu.SemaphoreType.DMA,
        ],
    )(uv, iv)


def kernel(users, items, user_emb, item_emb):
    return _mf_dot(users.astype(jnp.int32), items.astype(jnp.int32),
                   user_emb.T, item_emb.T)
